# Initial kernel scaffold; baseline (speedup 1.0000x reference)
import jax
import jax.numpy as jnp
from jax.experimental import pallas as pl


def kernel(x, edge_index, batch, W1, b1, W2, b2, W3, b3, g1, be1, g2, be2, g3, be3, sW1, sb1, sW2, sb2, aW1, ab1, aW2, ab2):
    raise NotImplementedError("write your pallas kernel here")


# speedup vs baseline: 8.8604x; 8.8604x over previous
import functools

import jax
import jax.numpy as jnp
from jax import lax
from jax.experimental import pallas as pl
from jax.experimental.pallas import tpu as pltpu
from jax.experimental.pallas import tpu_sc as plsc

N = 50000
NP = 51200
BR = 1024
G = NP // BR
E = 800000
CHUNK = 128
NCH = 6272
EP = NCH * CHUNK
H = 64
BN_EPS = 1e-5

_SC_TILES = 16
_RPT = NP // _SC_TILES


def _sc_mesh():
    return plsc.VectorSubcoreMesh(core_axis_name="c", subcore_axis_name="s")


_SC_PARAMS = pltpu.CompilerParams(use_tc_tiling_on_sc=False)


def _make_deg_kernel():
    per_worker = NCH // 32

    @functools.partial(
        pl.kernel,
        out_type=jax.ShapeDtypeStruct((NP, 2, 16), jnp.float32),
        mesh=_sc_mesh(),
        compiler_params=_SC_PARAMS,
        scratch_types=[
            pltpu.VMEM((1, CHUNK), jnp.int32),
            pltpu.VMEM((CHUNK, 16), jnp.float32),
            pltpu.VMEM_SHARED((NP, 16), jnp.float32),
        ],
    )
    def deg_kernel(dst_hbm, ones_hbm, zeros_hbm, out_hbm, dst_v, ones_v, acc):
        c = lax.axis_index("c")
        s = lax.axis_index("s")
        r0 = s * _RPT
        pltpu.sync_copy(zeros_hbm.at[pl.ds(r0, _RPT)], acc.at[pl.ds(r0, _RPT)])
        pltpu.sync_copy(ones_hbm, ones_v)
        plsc.subcore_barrier()
        base = (c * _SC_TILES + s) * per_worker

        def body(j, carry):
            ch = base + j
            pltpu.sync_copy(dst_hbm.at[pl.ds(ch, 1)], dst_v)
            pltpu.sync_copy(ones_v, acc.at[dst_v.at[0]], add=True)
            return carry

        lax.fori_loop(0, per_worker, body, 0)
        plsc.subcore_barrier()
        pltpu.sync_copy(acc.at[pl.ds(r0, _RPT)], out_hbm.at[pl.ds(r0, _RPT), c])

    return deg_kernel


def _make_scatter_kernel(w):
    per_sub = NCH // _SC_TILES

    @functools.partial(
        pl.kernel,
        out_type=jax.ShapeDtypeStruct((NP, 2, w), jnp.float32),
        mesh=_sc_mesh(),
        compiler_params=_SC_PARAMS,
        scratch_types=[
            pltpu.VMEM((CHUNK,), jnp.int32),
            pltpu.VMEM((CHUNK,), jnp.int32),
            pltpu.VMEM((1, CHUNK), jnp.int32),
            pltpu.VMEM((CHUNK, w), jnp.float32),
            pltpu.VMEM_SHARED((NP, w), jnp.float32),
            pltpu.SemaphoreType.DMA,
        ],
    )
    def scatter_kernel(src2_hbm, dst_hbm, table_hbm, zeros_hbm, out_hbm,
                       src_v, idx_v, dst_v, rows_v, acc, sem):
        c = lax.axis_index("c")
        s = lax.axis_index("s")
        r0 = s * _RPT
        pltpu.sync_copy(zeros_hbm.at[pl.ds(r0, _RPT)], acc.at[pl.ds(r0, _RPT)])
        plsc.subcore_barrier()
        base = s * per_sub

        def body(j, carry):
            ch = base + j
            pltpu.sync_copy(src2_hbm.at[ch], src_v)
            pltpu.sync_copy(dst_hbm.at[pl.ds(ch, 1)], dst_v)
            for t in range(CHUNK // 16):
                sl = pl.ds(t * 16, 16)
                idx_v[sl] = src_v[sl] + c
            pltpu.async_copy(table_hbm.at[idx_v], rows_v, sem).wait()
            pltpu.sync_copy(rows_v, acc.at[dst_v.at[0]], add=True)
            return carry

        lax.fori_loop(0, per_sub, body, 0)
        plsc.subcore_barrier()
        pltpu.sync_copy(acc.at[pl.ds(r0, _RPT)], out_hbm.at[pl.ds(r0, _RPT), c])

    return scatter_kernel


def _prep_body(deg_ref, x_ref, dinv_ref, xs_ref):
    d = deg_ref[:, 0:1] + deg_ref[:, 16:17] + 1.0
    dv = lax.rsqrt(d)
    dinv_ref[...] = dv
    xs_ref[...] = dv * x_ref[...]


def _tc_prep(degcnt2, x_pad):
    return pl.pallas_call(
        _prep_body,
        grid=(G,),
        in_specs=[
            pl.BlockSpec((BR, 32), lambda i: (i, 0)),
            pl.BlockSpec((BR, 32), lambda i: (i, 0)),
        ],
        out_specs=[
            pl.BlockSpec((BR, 1), lambda i: (i, 0)),
            pl.BlockSpec((BR, 32), lambda i: (i, 0)),
        ],
        out_shape=[
            jax.ShapeDtypeStruct((NP, 1), jnp.float32),
            jax.ShapeDtypeStruct((NP, 32), jnp.float32),
        ],
    )(degcnt2, x_pad)


def _layer_body(S_ref, xs_ref, dinv_ref, W_ref, scv_ref, shv_ref, out_ref):
    t = S_ref[...] + xs_ref[...]
    z = jnp.dot(t, W_ref[...], preferred_element_type=jnp.float32)
    dv = dinv_ref[...]
    a = jnp.maximum(dv * z * scv_ref[...] + shv_ref[...], 0.0)
    out_ref[...] = dv * a


def _tc_layer(S2w, xs_prev, dinv, Wp, scv, shv):
    win = Wp.shape[0]
    return pl.pallas_call(
        _layer_body,
        grid=(G,),
        in_specs=[
            pl.BlockSpec((BR, win), lambda i: (i, 0)),
            pl.BlockSpec((BR, win), lambda i: (i, 0)),
            pl.BlockSpec((BR, 1), lambda i: (i, 0)),
            pl.BlockSpec((win, H), lambda i: (0, 0)),
            pl.BlockSpec((1, H), lambda i: (0, 0)),
            pl.BlockSpec((1, H), lambda i: (0, 0)),
        ],
        out_specs=pl.BlockSpec((BR, H), lambda i: (i, 0)),
        out_shape=jax.ShapeDtypeStruct((NP, H), jnp.float32),
    )(S2w, xs_prev, dinv, Wp, scv, shv)


def _final_body(S_ref, xs_ref, dinv_ref, W_ref, scv_ref, shv_ref,
                sW1_ref, sb1_ref, sW2_ref, sb2_ref,
                aW1_ref, ab1_ref, aW2_ref, ab2_ref,
                score_ref, act_ref, acc_ref):
    i = pl.program_id(0)
    t = S_ref[...] + xs_ref[...]
    z = jnp.dot(t, W_ref[...], preferred_element_type=jnp.float32)
    a = jnp.maximum(dinv_ref[...] * z * scv_ref[...] + shv_ref[...], 0.0)
    row = i * BR + lax.broadcasted_iota(jnp.int32, (BR, 1), 0)
    a = jnp.where(row < N, a, 0.0)

    @pl.when(i == 0)
    def _():
        acc_ref[...] = jnp.zeros_like(acc_ref)

    acc_ref[...] += jnp.sum(a, axis=0, keepdims=True)

    @pl.when(i == G - 1)
    def _():
        emb = acc_ref[...] * (1.0 / N)
        h1 = jnp.maximum(
            jnp.dot(emb, sW1_ref[...], preferred_element_type=jnp.float32)
            + sb1_ref[...], 0.0)
        sc = jnp.dot(h1, sW2_ref[...], preferred_element_type=jnp.float32) \
            + sb2_ref[...]
        score_ref[...] = 1.0 / (1.0 + jnp.exp(-sc))
        h2 = jnp.maximum(
            jnp.dot(emb, aW1_ref[...], preferred_element_type=jnp.float32)
            + ab1_ref[...], 0.0)
        act_ref[...] = jnp.dot(h2, aW2_ref[...],
                               preferred_element_type=jnp.float32) + ab2_ref[...]


def _tc_final(S2w, xs_prev, dinv, Wp, scv, shv, sW1, sb1, sW2, sb2,
              aW1, ab1, aW2, ab2):
    full = lambda r, c: pl.BlockSpec((r, c), lambda i: (0, 0))
    return pl.pallas_call(
        _final_body,
        grid=(G,),
        in_specs=[
            pl.BlockSpec((BR, H), lambda i: (i, 0)),
            pl.BlockSpec((BR, H), lambda i: (i, 0)),
            pl.BlockSpec((BR, 1), lambda i: (i, 0)),
            full(H, H), full(1, H), full(1, H),
            full(H, 32), full(1, 32), full(32, 1), full(1, 1),
            full(H, 32), full(1, 32), full(32, 4), full(1, 4),
        ],
        out_specs=[full(1, 1), full(1, 4)],
        out_shape=[
            jax.ShapeDtypeStruct((1, 1), jnp.float32),
            jax.ShapeDtypeStruct((1, 4), jnp.float32),
        ],
        scratch_shapes=[pltpu.VMEM((1, H), jnp.float32)],
    )(S2w, xs_prev, dinv, Wp, scv, shv, sW1, sb1, sW2, sb2,
      aW1, ab1, aW2, ab2)


_deg_kernel = _make_deg_kernel()
_scatter16 = _make_scatter_kernel(16)
_scatter32 = _make_scatter_kernel(32)


def kernel(x, edge_index, batch, W1, b1, W2, b2, W3, b3, g1, be1, g2, be2,
           g3, be3, sW1, sb1, sW2, sb2, aW1, ab1, aW2, ab2):
    f32 = jnp.float32
    src = edge_index[0].astype(jnp.int32)
    dst = edge_index[1].astype(jnp.int32)

    npad = EP - E
    pad_i = jnp.arange(npad, dtype=jnp.int32)
    src_p = jnp.concatenate([src, pad_i % N])
    dst_p = jnp.concatenate([dst, N + (pad_i % (NP - N - 1))])
    src2 = (src_p * 2).reshape(NCH, CHUNK)
    dstc = dst_p.reshape(NCH, CHUNK)

    ones16 = jnp.ones((CHUNK, 16), f32)
    zeros16 = jnp.zeros((NP, 16), f32)
    zeros32 = jnp.zeros((NP, 32), f32)

    q = 1.0 / jnp.sqrt(1.0 + BN_EPS)
    scv1 = (g1 * q).reshape(1, H)
    shv1 = (b1 * g1 * q + be1).reshape(1, H)
    scv2 = (g2 * q).reshape(1, H)
    shv2 = (b2 * g2 * q + be2).reshape(1, H)
    scv3 = (g3 * q).reshape(1, H)
    shv3 = (b3 * g3 * q + be3).reshape(1, H)
    W1p = jnp.zeros((32, H), f32).at[:20].set(W1)

    x_pad = jnp.zeros((NP, 32), f32).at[:N, :20].set(x)

    degcnt = _deg_kernel(dstc, ones16, zeros16)
    dinv, xs1 = _tc_prep(degcnt.reshape(NP, 32), x_pad)

    S1 = _scatter16(src2, dstc, xs1.reshape(2 * NP, 16), zeros16)
    xs2 = _tc_layer(S1.reshape(NP, 32), xs1, dinv, W1p, scv1, shv1)

    S2 = _scatter32(src2, dstc, xs2.reshape(2 * NP, 32), zeros32)
    xs3 = _tc_layer(S2.reshape(NP, H), xs2, dinv, W2, scv2, shv2)

    S3 = _scatter32(src2, dstc, xs3.reshape(2 * NP, 32), zeros32)
    score, actions = _tc_final(
        S3.reshape(NP, H), xs3, dinv, W3, scv3, shv3,
        sW1, sb1.reshape(1, 32), sW2, sb2.reshape(1, 1),
        aW1, ab1.reshape(1, 32), aW2, ab2.reshape(1, 4))
    return (score, actions)


# speedup vs baseline: 14.6053x; 1.6484x over previous
import functools

import jax
import jax.numpy as jnp
from jax import lax
from jax.experimental import pallas as pl
from jax.experimental.pallas import tpu as pltpu
from jax.experimental.pallas import tpu_sc as plsc

N = 50000
NP = 51200
BR = 1024
G = NP // BR
E = 800000
CHUNK = 128
NCH = 6272
EP = NCH * CHUNK
H = 64
BN_EPS = 1e-5

_SC_TILES = 16
_RPT = NP // _SC_TILES


def _sc_mesh():
    return plsc.VectorSubcoreMesh(core_axis_name="c", subcore_axis_name="s")


_SC_PARAMS = pltpu.CompilerParams(use_tc_tiling_on_sc=False)


def _make_deg_kernel():
    per_worker = NCH // 32

    @functools.partial(
        pl.kernel,
        out_type=jax.ShapeDtypeStruct((NP, 2, 16), jnp.float32),
        mesh=_sc_mesh(),
        compiler_params=_SC_PARAMS,
        scratch_types=[
            pltpu.VMEM((per_worker, CHUNK), jnp.int32),
            pltpu.VMEM((CHUNK, 16), jnp.float32),
            pltpu.VMEM_SHARED((NP, 16), jnp.float32),
        ],
    )
    def deg_kernel(dst_hbm, ones_hbm, zeros_hbm, out_hbm, dst_v, ones_v, acc):
        c = lax.axis_index("c")
        s = lax.axis_index("s")
        r0 = s * _RPT
        pltpu.sync_copy(zeros_hbm.at[pl.ds(r0, _RPT)], acc.at[pl.ds(r0, _RPT)])
        pltpu.sync_copy(dst_hbm.at[c * _SC_TILES + s], dst_v)
        pltpu.sync_copy(ones_hbm, ones_v)
        plsc.subcore_barrier()

        def body(j, carry):
            pltpu.sync_copy(ones_v, acc.at[dst_v.at[j]], add=True)
            return carry

        lax.fori_loop(0, per_worker, body, 0)
        plsc.subcore_barrier()
        pltpu.sync_copy(acc.at[pl.ds(r0, _RPT)], out_hbm.at[pl.ds(r0, _RPT), c])

    return deg_kernel


def _make_scatter_kernel(w):
    per_sub = NCH // _SC_TILES
    BC = 28
    NB = per_sub // BC
    halfb = BC // 2

    @functools.partial(
        pl.kernel,
        out_type=jax.ShapeDtypeStruct((NP, 2, w), jnp.float32),
        mesh=_sc_mesh(),
        compiler_params=_SC_PARAMS,
        scratch_types=[
            pltpu.VMEM((BC * CHUNK,), jnp.int32),
            pltpu.VMEM((BC * CHUNK,), jnp.int32),
            pltpu.VMEM((BC, CHUNK), jnp.int32),
            pltpu.VMEM((BC, CHUNK), jnp.int32),
            pltpu.VMEM((CHUNK, w), jnp.float32),
            pltpu.VMEM((CHUNK, w), jnp.float32),
            pltpu.VMEM_SHARED((NP, w), jnp.float32),
            pltpu.SemaphoreType.DMA,
            pltpu.SemaphoreType.DMA,
            pltpu.SemaphoreType.DMA,
            pltpu.SemaphoreType.DMA,
        ],
    )
    def scatter_kernel(src2_hbm, dst_hbm, table_hbm, zeros_hbm, out_hbm,
                       srcA, srcB, dstA, dstB, rows0, rows1, acc,
                       sem0, sem1, semA, semB):
        c = lax.axis_index("c")
        s = lax.axis_index("s")
        r0 = s * _RPT
        pltpu.sync_copy(zeros_hbm.at[pl.ds(r0, _RPT)], acc.at[pl.ds(r0, _RPT)])

        src_bufs = (srcA, srcB)
        dst_bufs = (dstA, dstB)
        idx_sems = (semA, semB)

        def idx_srcs(b):
            return (src2_hbm.at[c, s, pl.ds(b * BC * CHUNK, BC * CHUNK)],
                    dst_hbm.at[s, b])

        def fire_idx(b, p):
            hs, hd = idx_srcs(b)
            pltpu.async_copy(hs, src_bufs[p], idx_sems[p])
            pltpu.async_copy(hd, dst_bufs[p], idx_sems[p])

        def wait_idx(b, p):
            hs, hd = idx_srcs(b)
            pltpu.make_async_copy(hs, src_bufs[p], idx_sems[p]).wait()
            pltpu.make_async_copy(hd, dst_bufs[p], idx_sems[p]).wait()

        fire_idx(0, 0)
        plsc.subcore_barrier()

        for b in range(NB):
            p = b % 2
            src_v, dst_v = src_bufs[p], dst_bufs[p]
            wait_idx(b, p)
            if b + 1 < NB:
                fire_idx(b + 1, 1 - p)

            def g_slice(j):
                return src_v.at[pl.ds(j * CHUNK, CHUNK)]

            pltpu.async_copy(table_hbm.at[g_slice(0)], rows0, sem0)

            def body(jj, carry):
                j = 2 * jj
                pltpu.make_async_copy(
                    table_hbm.at[g_slice(j)], rows0, sem0).wait()
                pltpu.async_copy(table_hbm.at[g_slice(j + 1)], rows1, sem1)
                pltpu.sync_copy(rows0, acc.at[dst_v.at[j]], add=True)
                pltpu.make_async_copy(
                    table_hbm.at[g_slice(j)], rows1, sem1).wait()

                @pl.when(jj < halfb - 1)
                def _():
                    pltpu.async_copy(table_hbm.at[g_slice(j + 2)], rows0, sem0)

                pltpu.sync_copy(rows1, acc.at[dst_v.at[j + 1]], add=True)
                return carry

            lax.fori_loop(0, halfb, body, 0)

        plsc.subcore_barrier()
        pltpu.sync_copy(acc.at[pl.ds(r0, _RPT)], out_hbm.at[pl.ds(r0, _RPT), c])

    return scatter_kernel


def _prep_body(deg_ref, x_ref, dinv_ref, xs_ref):
    d = deg_ref[:, 0:1] + deg_ref[:, 16:17] + 1.0
    dv = lax.rsqrt(d)
    dinv_ref[...] = dv
    xs_ref[...] = dv * x_ref[...]


def _tc_prep(degcnt2, x_pad):
    return pl.pallas_call(
        _prep_body,
        grid=(G,),
        in_specs=[
            pl.BlockSpec((BR, 32), lambda i: (i, 0)),
            pl.BlockSpec((BR, 32), lambda i: (i, 0)),
        ],
        out_specs=[
            pl.BlockSpec((BR, 1), lambda i: (i, 0)),
            pl.BlockSpec((BR, 32), lambda i: (i, 0)),
        ],
        out_shape=[
            jax.ShapeDtypeStruct((NP, 1), jnp.float32),
            jax.ShapeDtypeStruct((NP, 32), jnp.float32),
        ],
    )(degcnt2, x_pad)


def _layer_body(S_ref, xs_ref, dinv_ref, W_ref, scv_ref, shv_ref, out_ref):
    t = S_ref[...] + xs_ref[...]
    z = jnp.dot(t, W_ref[...], preferred_element_type=jnp.float32)
    dv = dinv_ref[...]
    a = jnp.maximum(dv * z * scv_ref[...] + shv_ref[...], 0.0)
    out_ref[...] = dv * a


def _tc_layer(S2w, xs_prev, dinv, Wp, scv, shv):
    win = Wp.shape[0]
    return pl.pallas_call(
        _layer_body,
        grid=(G,),
        in_specs=[
            pl.BlockSpec((BR, win), lambda i: (i, 0)),
            pl.BlockSpec((BR, win), lambda i: (i, 0)),
            pl.BlockSpec((BR, 1), lambda i: (i, 0)),
            pl.BlockSpec((win, H), lambda i: (0, 0)),
            pl.BlockSpec((1, H), lambda i: (0, 0)),
            pl.BlockSpec((1, H), lambda i: (0, 0)),
        ],
        out_specs=pl.BlockSpec((BR, H), lambda i: (i, 0)),
        out_shape=jax.ShapeDtypeStruct((NP, H), jnp.float32),
    )(S2w, xs_prev, dinv, Wp, scv, shv)


def _final_body(S_ref, xs_ref, dinv_ref, W_ref, scv_ref, shv_ref,
                sW1_ref, sb1_ref, sW2_ref, sb2_ref,
                aW1_ref, ab1_ref, aW2_ref, ab2_ref,
                score_ref, act_ref, acc_ref):
    i = pl.program_id(0)
    t = S_ref[...] + xs_ref[...]
    z = jnp.dot(t, W_ref[...], preferred_element_type=jnp.float32)
    a = jnp.maximum(dinv_ref[...] * z * scv_ref[...] + shv_ref[...], 0.0)
    row = i * BR + lax.broadcasted_iota(jnp.int32, (BR, 1), 0)
    a = jnp.where(row < N, a, 0.0)

    @pl.when(i == 0)
    def _():
        acc_ref[...] = jnp.zeros_like(acc_ref)

    acc_ref[...] += jnp.sum(a, axis=0, keepdims=True)

    @pl.when(i == G - 1)
    def _():
        emb = acc_ref[...] * (1.0 / N)
        h1 = jnp.maximum(
            jnp.dot(emb, sW1_ref[...], preferred_element_type=jnp.float32)
            + sb1_ref[...], 0.0)
        sc = jnp.dot(h1, sW2_ref[...], preferred_element_type=jnp.float32) \
            + sb2_ref[...]
        score_ref[...] = 1.0 / (1.0 + jnp.exp(-sc))
        h2 = jnp.maximum(
            jnp.dot(emb, aW1_ref[...], preferred_element_type=jnp.float32)
            + ab1_ref[...], 0.0)
        act_ref[...] = jnp.dot(h2, aW2_ref[...],
                               preferred_element_type=jnp.float32) + ab2_ref[...]


def _tc_final(S2w, xs_prev, dinv, Wp, scv, shv, sW1, sb1, sW2, sb2,
              aW1, ab1, aW2, ab2):
    full = lambda r, c: pl.BlockSpec((r, c), lambda i: (0, 0))
    return pl.pallas_call(
        _final_body,
        grid=(G,),
        in_specs=[
            pl.BlockSpec((BR, H), lambda i: (i, 0)),
            pl.BlockSpec((BR, H), lambda i: (i, 0)),
            pl.BlockSpec((BR, 1), lambda i: (i, 0)),
            full(H, H), full(1, H), full(1, H),
            full(H, 32), full(1, 32), full(32, 1), full(1, 1),
            full(H, 32), full(1, 32), full(32, 4), full(1, 4),
        ],
        out_specs=[full(1, 1), full(1, 4)],
        out_shape=[
            jax.ShapeDtypeStruct((1, 1), jnp.float32),
            jax.ShapeDtypeStruct((1, 4), jnp.float32),
        ],
        scratch_shapes=[pltpu.VMEM((1, H), jnp.float32)],
    )(S2w, xs_prev, dinv, Wp, scv, shv, sW1, sb1, sW2, sb2,
      aW1, ab1, aW2, ab2)


_deg_kernel = _make_deg_kernel()
_scatter16 = _make_scatter_kernel(16)
_scatter32 = _make_scatter_kernel(32)


def kernel(x, edge_index, batch, W1, b1, W2, b2, W3, b3, g1, be1, g2, be2,
           g3, be3, sW1, sb1, sW2, sb2, aW1, ab1, aW2, ab2):
    f32 = jnp.float32
    src = edge_index[0].astype(jnp.int32)
    dst = edge_index[1].astype(jnp.int32)

    npad = EP - E
    pad_i = jnp.arange(npad, dtype=jnp.int32)
    src_p = jnp.concatenate([src, pad_i % N])
    dst_p = jnp.concatenate([dst, N + (pad_i % (NP - N - 1))])
    src2_flat = src_p * 2
    src2 = jnp.stack([src2_flat, src2_flat + 1]).reshape(
        2, _SC_TILES, (NCH // _SC_TILES) * CHUNK)
    dstc = dst_p.reshape(_SC_TILES, (NCH // _SC_TILES) // 28, 28, CHUNK)
    dstc_deg = dst_p.reshape(32, NCH // 32, CHUNK)

    ones16 = jnp.ones((CHUNK, 16), f32)
    zeros16 = jnp.zeros((NP, 16), f32)
    zeros32 = jnp.zeros((NP, 32), f32)

    q = 1.0 / jnp.sqrt(1.0 + BN_EPS)
    scv1 = (g1 * q).reshape(1, H)
    shv1 = (b1 * g1 * q + be1).reshape(1, H)
    scv2 = (g2 * q).reshape(1, H)
    shv2 = (b2 * g2 * q + be2).reshape(1, H)
    scv3 = (g3 * q).reshape(1, H)
    shv3 = (b3 * g3 * q + be3).reshape(1, H)
    W1p = jnp.zeros((32, H), f32).at[:20].set(W1)

    x_pad = jnp.zeros((NP, 32), f32).at[:N, :20].set(x)

    degcnt = _deg_kernel(dstc_deg, ones16, zeros16)
    dinv, xs1 = _tc_prep(degcnt.reshape(NP, 32), x_pad)

    S1 = _scatter16(src2, dstc, xs1.reshape(2 * NP, 16), zeros16)
    xs2 = _tc_layer(S1.reshape(NP, 32), xs1, dinv, W1p, scv1, shv1)

    S2 = _scatter32(src2, dstc, xs2.reshape(2 * NP, 32), zeros32)
    xs3 = _tc_layer(S2.reshape(NP, H), xs2, dinv, W2, scv2, shv2)

    S3 = _scatter32(src2, dstc, xs3.reshape(2 * NP, 32), zeros32)
    score, actions = _tc_final(
        S3.reshape(NP, H), xs3, dinv, W3, scv3, shv3,
        sW1, sb1.reshape(1, 32), sW2, sb2.reshape(1, 1),
        aW1, ab1.reshape(1, 32), aW2, ab2.reshape(1, 4))
    return (score, actions)


# speedup vs baseline: 19.5338x; 1.3374x over previous
import functools

import jax
import jax.numpy as jnp
from jax import lax
from jax.experimental import pallas as pl
from jax.experimental.pallas import tpu as pltpu
from jax.experimental.pallas import tpu_sc as plsc

N = 50000
NP = 51200
BR = 1024
G = NP // BR
E = 800000
CHUNK = 128
NCH = 6272
EP = NCH * CHUNK
H = 64
BN_EPS = 1e-5

_SC_TILES = 16
_RPT = NP // _SC_TILES


def _sc_mesh():
    return plsc.VectorSubcoreMesh(core_axis_name="c", subcore_axis_name="s")


_SC_PARAMS = pltpu.CompilerParams(use_tc_tiling_on_sc=False)


def _make_deg_kernel():
    per_worker = NCH // 32

    @functools.partial(
        pl.kernel,
        out_type=jax.ShapeDtypeStruct((NP, 2, 16), jnp.float32),
        mesh=_sc_mesh(),
        compiler_params=_SC_PARAMS,
        scratch_types=[
            pltpu.VMEM((per_worker, CHUNK), jnp.int32),
            pltpu.VMEM((CHUNK, 16), jnp.float32),
            pltpu.VMEM_SHARED((NP, 16), jnp.float32),
        ],
    )
    def deg_kernel(dst_hbm, ones_hbm, zeros_hbm, out_hbm, dst_v, ones_v, acc):
        c = lax.axis_index("c")
        s = lax.axis_index("s")
        r0 = s * _RPT
        pltpu.sync_copy(zeros_hbm.at[pl.ds(r0, _RPT)], acc.at[pl.ds(r0, _RPT)])
        pltpu.sync_copy(dst_hbm.at[c * _SC_TILES + s], dst_v)
        pltpu.sync_copy(ones_hbm, ones_v)
        plsc.subcore_barrier()

        def body(j, carry):
            pltpu.sync_copy(ones_v, acc.at[dst_v.at[j]], add=True)
            return carry

        lax.fori_loop(0, per_worker, body, 0)
        plsc.subcore_barrier()
        pltpu.sync_copy(acc.at[pl.ds(r0, _RPT)], out_hbm.at[pl.ds(r0, _RPT), c])

    return deg_kernel


def _make_scatter_kernel(w):
    per_sub = NCH // _SC_TILES
    BC = 28
    NB = per_sub // BC
    NQ = BC // 4

    @functools.partial(
        pl.kernel,
        out_type=jax.ShapeDtypeStruct((NP, 2, w), jnp.float32),
        mesh=_sc_mesh(),
        compiler_params=_SC_PARAMS,
        scratch_types=[
            pltpu.VMEM((BC * CHUNK,), jnp.int32),
            pltpu.VMEM((BC * CHUNK,), jnp.int32),
            pltpu.VMEM((BC, CHUNK), jnp.int32),
            [pltpu.VMEM((CHUNK, w), jnp.float32) for _ in range(4)],
            pltpu.VMEM_SHARED((NP, w), jnp.float32),
            [pltpu.SemaphoreType.DMA for _ in range(4)],
            [pltpu.SemaphoreType.DMA for _ in range(4)],
            pltpu.SemaphoreType.DMA,
            pltpu.SemaphoreType.DMA,
            pltpu.SemaphoreType.DMA,
        ],
    )
    def scatter_kernel(src2_hbm, dst_hbm, table_hbm, zeros_hbm, out_hbm,
                       srcA, srcB, dst_v, rows, acc, gsem, ssem,
                       semA, semB, semD):
        c = lax.axis_index("c")
        s = lax.axis_index("s")
        r0 = s * _RPT
        pltpu.sync_copy(zeros_hbm.at[pl.ds(r0, _RPT)], acc.at[pl.ds(r0, _RPT)])

        src_bufs = (srcA, srcB)
        idx_sems = (semA, semB)

        def src_hslice(b):
            return src2_hbm.at[c, s, pl.ds(b * BC * CHUNK, BC * CHUNK)]

        def fire_src(b, p):
            pltpu.async_copy(src_hslice(b), src_bufs[p], idx_sems[p])

        def wait_src(b, p):
            pltpu.make_async_copy(src_hslice(b), src_bufs[p],
                                  idx_sems[p]).wait()

        def fire_dst(b):
            pltpu.async_copy(dst_hbm.at[s, b], dst_v, semD)

        def wait_dst(b):
            pltpu.make_async_copy(dst_hbm.at[s, b], dst_v, semD).wait()

        def g_slice(buf, j):
            return buf.at[pl.ds(j * CHUNK, CHUNK)]

        def fire_g(buf, j, k):
            pltpu.async_copy(table_hbm.at[g_slice(buf, j)], rows[k], gsem[k])

        def wait_g(buf, j, k):
            pltpu.make_async_copy(table_hbm.at[g_slice(buf, j)], rows[k],
                                  gsem[k]).wait()

        fire_src(0, 0)
        fire_dst(0)
        plsc.subcore_barrier()
        wait_src(0, 0)
        for k in range(4):
            fire_g(srcA, k, k)

        for b in range(NB):
            p = b % 2
            src_v = src_bufs[p]
            if b + 1 < NB:
                fire_src(b + 1, 1 - p)
            wait_dst(b)

            def body(ii, carry):
                j = 4 * ii
                for k in range(4):
                    wait_g(src_v, j + k, k)
                    pltpu.async_copy(rows[k], acc.at[dst_v.at[j + k]],
                                     ssem[k], add=True)
                for k in range(4):
                    pltpu.make_async_copy(rows[k], acc.at[dst_v.at[j + k]],
                                          ssem[k]).wait()

                    @pl.when(ii < NQ - 1)
                    def _():
                        fire_g(src_v, j + 4 + k, k)

                    if b + 1 < NB:
                        @pl.when(ii == NQ - 1)
                        def _():
                            if k == 0:
                                wait_src(b + 1, 1 - p)
                            fire_g(src_bufs[1 - p], k, k)

                return carry

            lax.fori_loop(0, NQ, body, 0)
            if b + 1 < NB:
                fire_dst(b + 1)

        plsc.subcore_barrier()
        pltpu.sync_copy(acc.at[pl.ds(r0, _RPT)], out_hbm.at[pl.ds(r0, _RPT), c])

    return scatter_kernel


def _prep_body(deg_ref, x_ref, dinv_ref, xs_ref):
    d = deg_ref[:, 0:1] + deg_ref[:, 16:17] + 1.0
    dv = lax.rsqrt(d)
    dinv_ref[...] = dv
    xs_ref[...] = dv * x_ref[...]


def _tc_prep(degcnt2, x_pad):
    return pl.pallas_call(
        _prep_body,
        grid=(G,),
        in_specs=[
            pl.BlockSpec((BR, 32), lambda i: (i, 0)),
            pl.BlockSpec((BR, 32), lambda i: (i, 0)),
        ],
        out_specs=[
            pl.BlockSpec((BR, 1), lambda i: (i, 0)),
            pl.BlockSpec((BR, 32), lambda i: (i, 0)),
        ],
        out_shape=[
            jax.ShapeDtypeStruct((NP, 1), jnp.float32),
            jax.ShapeDtypeStruct((NP, 32), jnp.float32),
        ],
    )(degcnt2, x_pad)


def _layer_body(S_ref, xs_ref, dinv_ref, W_ref, scv_ref, shv_ref, out_ref):
    t = S_ref[...] + xs_ref[...]
    z = jnp.dot(t, W_ref[...], preferred_element_type=jnp.float32)
    dv = dinv_ref[...]
    a = jnp.maximum(dv * z * scv_ref[...] + shv_ref[...], 0.0)
    out_ref[...] = dv * a


def _tc_layer(S2w, xs_prev, dinv, Wp, scv, shv):
    win = Wp.shape[0]
    return pl.pallas_call(
        _layer_body,
        grid=(G,),
        in_specs=[
            pl.BlockSpec((BR, win), lambda i: (i, 0)),
            pl.BlockSpec((BR, win), lambda i: (i, 0)),
            pl.BlockSpec((BR, 1), lambda i: (i, 0)),
            pl.BlockSpec((win, H), lambda i: (0, 0)),
            pl.BlockSpec((1, H), lambda i: (0, 0)),
            pl.BlockSpec((1, H), lambda i: (0, 0)),
        ],
        out_specs=pl.BlockSpec((BR, H), lambda i: (i, 0)),
        out_shape=jax.ShapeDtypeStruct((NP, H), jnp.float32),
    )(S2w, xs_prev, dinv, Wp, scv, shv)


def _final_body(S_ref, xs_ref, dinv_ref, W_ref, scv_ref, shv_ref,
                sW1_ref, sb1_ref, sW2_ref, sb2_ref,
                aW1_ref, ab1_ref, aW2_ref, ab2_ref,
                score_ref, act_ref, acc_ref):
    i = pl.program_id(0)
    t = S_ref[...] + xs_ref[...]
    z = jnp.dot(t, W_ref[...], preferred_element_type=jnp.float32)
    a = jnp.maximum(dinv_ref[...] * z * scv_ref[...] + shv_ref[...], 0.0)
    row = i * BR + lax.broadcasted_iota(jnp.int32, (BR, 1), 0)
    a = jnp.where(row < N, a, 0.0)

    @pl.when(i == 0)
    def _():
        acc_ref[...] = jnp.zeros_like(acc_ref)

    acc_ref[...] += jnp.sum(a, axis=0, keepdims=True)

    @pl.when(i == G - 1)
    def _():
        emb = acc_ref[...] * (1.0 / N)
        h1 = jnp.maximum(
            jnp.dot(emb, sW1_ref[...], preferred_element_type=jnp.float32)
            + sb1_ref[...], 0.0)
        sc = jnp.dot(h1, sW2_ref[...], preferred_element_type=jnp.float32) \
            + sb2_ref[...]
        score_ref[...] = 1.0 / (1.0 + jnp.exp(-sc))
        h2 = jnp.maximum(
            jnp.dot(emb, aW1_ref[...], preferred_element_type=jnp.float32)
            + ab1_ref[...], 0.0)
        act_ref[...] = jnp.dot(h2, aW2_ref[...],
                               preferred_element_type=jnp.float32) + ab2_ref[...]


def _tc_final(S2w, xs_prev, dinv, Wp, scv, shv, sW1, sb1, sW2, sb2,
              aW1, ab1, aW2, ab2):
    full = lambda r, c: pl.BlockSpec((r, c), lambda i: (0, 0))
    return pl.pallas_call(
        _final_body,
        grid=(G,),
        in_specs=[
            pl.BlockSpec((BR, H), lambda i: (i, 0)),
            pl.BlockSpec((BR, H), lambda i: (i, 0)),
            pl.BlockSpec((BR, 1), lambda i: (i, 0)),
            full(H, H), full(1, H), full(1, H),
            full(H, 32), full(1, 32), full(32, 1), full(1, 1),
            full(H, 32), full(1, 32), full(32, 4), full(1, 4),
        ],
        out_specs=[full(1, 1), full(1, 4)],
        out_shape=[
            jax.ShapeDtypeStruct((1, 1), jnp.float32),
            jax.ShapeDtypeStruct((1, 4), jnp.float32),
        ],
        scratch_shapes=[pltpu.VMEM((1, H), jnp.float32)],
    )(S2w, xs_prev, dinv, Wp, scv, shv, sW1, sb1, sW2, sb2,
      aW1, ab1, aW2, ab2)


_deg_kernel = _make_deg_kernel()
_scatter16 = _make_scatter_kernel(16)
_scatter32 = _make_scatter_kernel(32)


def kernel(x, edge_index, batch, W1, b1, W2, b2, W3, b3, g1, be1, g2, be2,
           g3, be3, sW1, sb1, sW2, sb2, aW1, ab1, aW2, ab2):
    f32 = jnp.float32
    src = edge_index[0].astype(jnp.int32)
    dst = edge_index[1].astype(jnp.int32)

    npad = EP - E
    pad_i = jnp.arange(npad, dtype=jnp.int32)
    src_p = jnp.concatenate([src, pad_i % N])
    dst_p = jnp.concatenate([dst, N + (pad_i % (NP - N - 1))])
    src2_flat = src_p * 2
    src2 = jnp.stack([src2_flat, src2_flat + 1]).reshape(
        2, _SC_TILES, (NCH // _SC_TILES) * CHUNK)
    dstc = dst_p.reshape(_SC_TILES, (NCH // _SC_TILES) // 28, 28, CHUNK)
    dstc_deg = dst_p.reshape(32, NCH // 32, CHUNK)

    ones16 = jnp.ones((CHUNK, 16), f32)
    zeros16 = jnp.zeros((NP, 16), f32)
    zeros32 = jnp.zeros((NP, 32), f32)

    q = 1.0 / jnp.sqrt(1.0 + BN_EPS)
    scv1 = (g1 * q).reshape(1, H)
    shv1 = (b1 * g1 * q + be1).reshape(1, H)
    scv2 = (g2 * q).reshape(1, H)
    shv2 = (b2 * g2 * q + be2).reshape(1, H)
    scv3 = (g3 * q).reshape(1, H)
    shv3 = (b3 * g3 * q + be3).reshape(1, H)
    W1p = jnp.zeros((32, H), f32).at[:20].set(W1)

    x_pad = jnp.zeros((NP, 32), f32).at[:N, :20].set(x)

    degcnt = _deg_kernel(dstc_deg, ones16, zeros16)
    dinv, xs1 = _tc_prep(degcnt.reshape(NP, 32), x_pad)

    S1 = _scatter16(src2, dstc, xs1.reshape(2 * NP, 16), zeros16)
    xs2 = _tc_layer(S1.reshape(NP, 32), xs1, dinv, W1p, scv1, shv1)

    S2 = _scatter32(src2, dstc, xs2.reshape(2 * NP, 32), zeros32)
    xs3 = _tc_layer(S2.reshape(NP, H), xs2, dinv, W2, scv2, shv2)

    S3 = _scatter32(src2, dstc, xs3.reshape(2 * NP, 32), zeros32)
    score, actions = _tc_final(
        S3.reshape(NP, H), xs3, dinv, W3, scv3, shv3,
        sW1, sb1.reshape(1, 32), sW2, sb2.reshape(1, 1),
        aW1, ab1.reshape(1, 32), aW2, ab2.reshape(1, 4))
    return (score, actions)


# speedup vs baseline: 19.7467x; 1.0109x over previous
import functools

import jax
import jax.numpy as jnp
from jax import lax
from jax.experimental import pallas as pl
from jax.experimental.pallas import tpu as pltpu
from jax.experimental.pallas import tpu_sc as plsc

N = 50000
NP = 51200
BR = 1024
G = NP // BR
E = 800000
CHUNK = 128
NCH = 6272
EP = NCH * CHUNK
H = 64
BN_EPS = 1e-5

_SC_TILES = 16
_RPT = NP // _SC_TILES


def _sc_mesh():
    return plsc.VectorSubcoreMesh(core_axis_name="c", subcore_axis_name="s")


_SC_PARAMS = pltpu.CompilerParams(use_tc_tiling_on_sc=False)


def _make_deg_kernel():
    per_worker = NCH // 32

    @functools.partial(
        pl.kernel,
        out_type=jax.ShapeDtypeStruct((NP, 2, 16), jnp.float32),
        mesh=_sc_mesh(),
        compiler_params=_SC_PARAMS,
        scratch_types=[
            pltpu.VMEM((per_worker, CHUNK), jnp.int32),
            pltpu.VMEM((CHUNK, 16), jnp.float32),
            pltpu.VMEM_SHARED((NP, 16), jnp.float32),
        ],
    )
    def deg_kernel(dst_hbm, ones_hbm, zeros_hbm, out_hbm, dst_v, ones_v, acc):
        c = lax.axis_index("c")
        s = lax.axis_index("s")
        r0 = s * _RPT
        pltpu.sync_copy(zeros_hbm.at[pl.ds(r0, _RPT)], acc.at[pl.ds(r0, _RPT)])
        pltpu.sync_copy(dst_hbm.at[c * _SC_TILES + s], dst_v)
        pltpu.sync_copy(ones_hbm, ones_v)
        plsc.subcore_barrier()

        def body(j, carry):
            pltpu.sync_copy(ones_v, acc.at[dst_v.at[j]], add=True)
            return carry

        lax.fori_loop(0, per_worker, body, 0)
        plsc.subcore_barrier()
        pltpu.sync_copy(acc.at[pl.ds(r0, _RPT)], out_hbm.at[pl.ds(r0, _RPT), c])

    return deg_kernel


def _make_scatter_kernel(w):
    per_sub = NCH // _SC_TILES
    BC = 28
    NB = per_sub // BC
    NQ = BC // 4

    @functools.partial(
        pl.kernel,
        out_type=jax.ShapeDtypeStruct((NP, 2, w), jnp.float32),
        mesh=_sc_mesh(),
        compiler_params=_SC_PARAMS,
        scratch_types=[
            pltpu.VMEM((BC * CHUNK,), jnp.int32),
            pltpu.VMEM((BC * CHUNK,), jnp.int32),
            pltpu.VMEM((BC, CHUNK), jnp.int32),
            [pltpu.VMEM((CHUNK, w), jnp.float32) for _ in range(4)],
            pltpu.VMEM_SHARED((NP, w), jnp.float32),
            [pltpu.SemaphoreType.DMA for _ in range(4)],
            [pltpu.SemaphoreType.DMA for _ in range(4)],
            pltpu.SemaphoreType.DMA,
            pltpu.SemaphoreType.DMA,
            pltpu.SemaphoreType.DMA,
        ],
    )
    def scatter_kernel(src2_hbm, dst_hbm, table_hbm, zeros_hbm, out_hbm,
                       srcA, srcB, dst_v, rows, acc, gsem, ssem,
                       semA, semB, semD):
        c = lax.axis_index("c")
        s = lax.axis_index("s")
        r0 = s * _RPT
        pltpu.sync_copy(zeros_hbm.at[pl.ds(r0, _RPT)], acc.at[pl.ds(r0, _RPT)])

        src_bufs = (srcA, srcB)
        idx_sems = (semA, semB)

        def src_hslice(b):
            return src2_hbm.at[c, s, pl.ds(b * BC * CHUNK, BC * CHUNK)]

        def fire_src(b, p):
            pltpu.async_copy(src_hslice(b), src_bufs[p], idx_sems[p])

        def wait_src(b, p):
            pltpu.make_async_copy(src_hslice(b), src_bufs[p],
                                  idx_sems[p]).wait()

        def fire_dst(b):
            pltpu.async_copy(dst_hbm.at[s, b], dst_v, semD)

        def wait_dst(b):
            pltpu.make_async_copy(dst_hbm.at[s, b], dst_v, semD).wait()

        def g_slice(buf, j):
            return buf.at[pl.ds(j * CHUNK, CHUNK)]

        def fire_g(buf, j, k):
            pltpu.async_copy(table_hbm.at[g_slice(buf, j)], rows[k], gsem[k])

        def wait_g(buf, j, k):
            pltpu.make_async_copy(table_hbm.at[g_slice(buf, j)], rows[k],
                                  gsem[k]).wait()

        fire_src(0, 0)
        fire_dst(0)
        plsc.subcore_barrier()
        wait_src(0, 0)
        for k in range(4):
            fire_g(srcA, k, k)

        for b in range(NB):
            p = b % 2
            src_v = src_bufs[p]
            if b + 1 < NB:
                fire_src(b + 1, 1 - p)
            wait_dst(b)

            def wait_s(j, k):
                pltpu.make_async_copy(rows[k], acc.at[dst_v.at[j]],
                                      ssem[k]).wait()

            def body(ii, carry):
                j = 4 * ii
                for k in range(4):
                    wait_g(src_v, j + k, k)
                    kp = (k + 3) % 4
                    if k == 0:
                        @pl.when(ii > 0)
                        def _():
                            fire_g(src_v, j + 3, kp)
                    else:
                        @pl.when(ii < NQ - 1)
                        def _():
                            fire_g(src_v, j + k + 3, kp)

                        if b + 1 < NB:
                            @pl.when(ii == NQ - 1)
                            def _():
                                if k == 1:
                                    wait_src(b + 1, 1 - p)
                                fire_g(src_bufs[1 - p], k - 1, kp)

                    pltpu.sync_copy(rows[k], acc.at[dst_v.at[j + k]],
                                    add=True)
                return carry

            lax.fori_loop(0, NQ, body, 0)
            if b + 1 < NB:
                fire_g(src_bufs[1 - p], 3, 3)
                fire_dst(b + 1)

        plsc.subcore_barrier()
        pltpu.sync_copy(acc.at[pl.ds(r0, _RPT)], out_hbm.at[pl.ds(r0, _RPT), c])

    return scatter_kernel


def _prep_body(deg_ref, x_ref, dinv_ref, xs_ref):
    d = deg_ref[:, 0:1] + deg_ref[:, 16:17] + 1.0
    dv = lax.rsqrt(d)
    dinv_ref[...] = dv
    xs_ref[...] = dv * x_ref[...]


def _tc_prep(degcnt2, x_pad):
    return pl.pallas_call(
        _prep_body,
        grid=(G,),
        in_specs=[
            pl.BlockSpec((BR, 32), lambda i: (i, 0)),
            pl.BlockSpec((BR, 32), lambda i: (i, 0)),
        ],
        out_specs=[
            pl.BlockSpec((BR, 1), lambda i: (i, 0)),
            pl.BlockSpec((BR, 32), lambda i: (i, 0)),
        ],
        out_shape=[
            jax.ShapeDtypeStruct((NP, 1), jnp.float32),
            jax.ShapeDtypeStruct((NP, 32), jnp.float32),
        ],
    )(degcnt2, x_pad)


def _layer_body(S_ref, xs_ref, dinv_ref, W_ref, scv_ref, shv_ref, out_ref):
    t = S_ref[...] + xs_ref[...]
    z = jnp.dot(t, W_ref[...], preferred_element_type=jnp.float32)
    dv = dinv_ref[...]
    a = jnp.maximum(dv * z * scv_ref[...] + shv_ref[...], 0.0)
    out_ref[...] = dv * a


def _tc_layer(S2w, xs_prev, dinv, Wp, scv, shv):
    win = Wp.shape[0]
    return pl.pallas_call(
        _layer_body,
        grid=(G,),
        in_specs=[
            pl.BlockSpec((BR, win), lambda i: (i, 0)),
            pl.BlockSpec((BR, win), lambda i: (i, 0)),
            pl.BlockSpec((BR, 1), lambda i: (i, 0)),
            pl.BlockSpec((win, H), lambda i: (0, 0)),
            pl.BlockSpec((1, H), lambda i: (0, 0)),
            pl.BlockSpec((1, H), lambda i: (0, 0)),
        ],
        out_specs=pl.BlockSpec((BR, H), lambda i: (i, 0)),
        out_shape=jax.ShapeDtypeStruct((NP, H), jnp.float32),
    )(S2w, xs_prev, dinv, Wp, scv, shv)


def _final_body(S_ref, xs_ref, dinv_ref, W_ref, scv_ref, shv_ref,
                sW1_ref, sb1_ref, sW2_ref, sb2_ref,
                aW1_ref, ab1_ref, aW2_ref, ab2_ref,
                score_ref, act_ref, acc_ref):
    i = pl.program_id(0)
    t = S_ref[...] + xs_ref[...]
    z = jnp.dot(t, W_ref[...], preferred_element_type=jnp.float32)
    a = jnp.maximum(dinv_ref[...] * z * scv_ref[...] + shv_ref[...], 0.0)
    row = i * BR + lax.broadcasted_iota(jnp.int32, (BR, 1), 0)
    a = jnp.where(row < N, a, 0.0)

    @pl.when(i == 0)
    def _():
        acc_ref[...] = jnp.zeros_like(acc_ref)

    acc_ref[...] += jnp.sum(a, axis=0, keepdims=True)

    @pl.when(i == G - 1)
    def _():
        emb = acc_ref[...] * (1.0 / N)
        h1 = jnp.maximum(
            jnp.dot(emb, sW1_ref[...], preferred_element_type=jnp.float32)
            + sb1_ref[...], 0.0)
        sc = jnp.dot(h1, sW2_ref[...], preferred_element_type=jnp.float32) \
            + sb2_ref[...]
        score_ref[...] = 1.0 / (1.0 + jnp.exp(-sc))
        h2 = jnp.maximum(
            jnp.dot(emb, aW1_ref[...], preferred_element_type=jnp.float32)
            + ab1_ref[...], 0.0)
        act_ref[...] = jnp.dot(h2, aW2_ref[...],
                               preferred_element_type=jnp.float32) + ab2_ref[...]


def _tc_final(S2w, xs_prev, dinv, Wp, scv, shv, sW1, sb1, sW2, sb2,
              aW1, ab1, aW2, ab2):
    full = lambda r, c: pl.BlockSpec((r, c), lambda i: (0, 0))
    return pl.pallas_call(
        _final_body,
        grid=(G,),
        in_specs=[
            pl.BlockSpec((BR, H), lambda i: (i, 0)),
            pl.BlockSpec((BR, H), lambda i: (i, 0)),
            pl.BlockSpec((BR, 1), lambda i: (i, 0)),
            full(H, H), full(1, H), full(1, H),
            full(H, 32), full(1, 32), full(32, 1), full(1, 1),
            full(H, 32), full(1, 32), full(32, 4), full(1, 4),
        ],
        out_specs=[full(1, 1), full(1, 4)],
        out_shape=[
            jax.ShapeDtypeStruct((1, 1), jnp.float32),
            jax.ShapeDtypeStruct((1, 4), jnp.float32),
        ],
        scratch_shapes=[pltpu.VMEM((1, H), jnp.float32)],
    )(S2w, xs_prev, dinv, Wp, scv, shv, sW1, sb1, sW2, sb2,
      aW1, ab1, aW2, ab2)


_deg_kernel = _make_deg_kernel()
_scatter16 = _make_scatter_kernel(16)
_scatter32 = _make_scatter_kernel(32)


def kernel(x, edge_index, batch, W1, b1, W2, b2, W3, b3, g1, be1, g2, be2,
           g3, be3, sW1, sb1, sW2, sb2, aW1, ab1, aW2, ab2):
    f32 = jnp.float32
    src = edge_index[0].astype(jnp.int32)
    dst = edge_index[1].astype(jnp.int32)

    npad = EP - E
    pad_i = jnp.arange(npad, dtype=jnp.int32)
    src_p = jnp.concatenate([src, pad_i % N])
    dst_p = jnp.concatenate([dst, N + (pad_i % (NP - N - 1))])
    src2_flat = src_p * 2
    src2 = jnp.stack([src2_flat, src2_flat + 1]).reshape(
        2, _SC_TILES, (NCH // _SC_TILES) * CHUNK)
    dstc = dst_p.reshape(_SC_TILES, (NCH // _SC_TILES) // 28, 28, CHUNK)
    dstc_deg = dst_p.reshape(32, NCH // 32, CHUNK)

    ones16 = jnp.ones((CHUNK, 16), f32)
    zeros16 = jnp.zeros((NP, 16), f32)
    zeros32 = jnp.zeros((NP, 32), f32)

    q = 1.0 / jnp.sqrt(1.0 + BN_EPS)
    scv1 = (g1 * q).reshape(1, H)
    shv1 = (b1 * g1 * q + be1).reshape(1, H)
    scv2 = (g2 * q).reshape(1, H)
    shv2 = (b2 * g2 * q + be2).reshape(1, H)
    scv3 = (g3 * q).reshape(1, H)
    shv3 = (b3 * g3 * q + be3).reshape(1, H)
    W1p = jnp.zeros((32, H), f32).at[:20].set(W1)

    x_pad = jnp.zeros((NP, 32), f32).at[:N, :20].set(x)

    degcnt = _deg_kernel(dstc_deg, ones16, zeros16)
    dinv, xs1 = _tc_prep(degcnt.reshape(NP, 32), x_pad)

    S1 = _scatter16(src2, dstc, xs1.reshape(2 * NP, 16), zeros16)
    xs2 = _tc_layer(S1.reshape(NP, 32), xs1, dinv, W1p, scv1, shv1)

    S2 = _scatter32(src2, dstc, xs2.reshape(2 * NP, 32), zeros32)
    xs3 = _tc_layer(S2.reshape(NP, H), xs2, dinv, W2, scv2, shv2)

    S3 = _scatter32(src2, dstc, xs3.reshape(2 * NP, 32), zeros32)
    score, actions = _tc_final(
        S3.reshape(NP, H), xs3, dinv, W3, scv3, shv3,
        sW1, sb1.reshape(1, 32), sW2, sb2.reshape(1, 1),
        aW1, ab1.reshape(1, 32), aW2, ab2.reshape(1, 4))
    return (score, actions)


# speedup vs baseline: 38.4933x; 1.9494x over previous
import functools

import jax
import jax.numpy as jnp
from jax import lax
from jax.experimental import pallas as pl
from jax.experimental.pallas import tpu as pltpu
from jax.experimental.pallas import tpu_sc as plsc

N = 50000
NP = 51200
BR = 1024
G = NP // BR
E = 800000
CHUNK = 128
NCH = 6272
EP = NCH * CHUNK
H = 64
BN_EPS = 1e-5

_SC_TILES = 16
_RPT = NP // _SC_TILES


def _sc_mesh():
    return plsc.VectorSubcoreMesh(core_axis_name="c", subcore_axis_name="s")


_SC_PARAMS = pltpu.CompilerParams(use_tc_tiling_on_sc=False)


def _make_deg_kernel():
    per_worker = NCH // 32

    @functools.partial(
        pl.kernel,
        out_type=jax.ShapeDtypeStruct((NP, 2, 16), jnp.float32),
        mesh=_sc_mesh(),
        compiler_params=_SC_PARAMS,
        scratch_types=[
            pltpu.VMEM((per_worker, CHUNK), jnp.int32),
            pltpu.VMEM((CHUNK, 16), jnp.float32),
            pltpu.VMEM_SHARED((NP, 16), jnp.float32),
        ],
    )
    def deg_kernel(dst_hbm, ones_hbm, zeros_hbm, out_hbm, dst_v, ones_v, acc):
        c = lax.axis_index("c")
        s = lax.axis_index("s")
        r0 = s * _RPT
        pltpu.sync_copy(zeros_hbm.at[pl.ds(r0, _RPT)], acc.at[pl.ds(r0, _RPT)])
        pltpu.sync_copy(dst_hbm.at[c * _SC_TILES + s], dst_v)
        pltpu.sync_copy(ones_hbm, ones_v)
        plsc.subcore_barrier()

        def body(j, carry):
            pltpu.sync_copy(ones_v, acc.at[dst_v.at[j]], add=True)
            return carry

        lax.fori_loop(0, per_worker, body, 0)
        plsc.subcore_barrier()
        pltpu.sync_copy(acc.at[pl.ds(r0, _RPT)], out_hbm.at[pl.ds(r0, _RPT), c])

    return deg_kernel


def _make_scatter_kernel(w):
    per_sub = NCH // _SC_TILES
    BC = 28
    NB = per_sub // BC
    NQ = BC // 4

    @functools.partial(
        pl.kernel,
        out_type=jax.ShapeDtypeStruct((NP, 2, w), jnp.float32),
        mesh=_sc_mesh(),
        compiler_params=_SC_PARAMS,
        scratch_types=[
            pltpu.VMEM((BC * CHUNK,), jnp.int32),
            pltpu.VMEM((BC * CHUNK,), jnp.int32),
            pltpu.VMEM((BC, CHUNK), jnp.int32),
            [pltpu.VMEM((CHUNK, w), jnp.float32) for _ in range(4)],
            pltpu.VMEM_SHARED((NP, w), jnp.float32),
            [pltpu.SemaphoreType.DMA for _ in range(4)],
            [pltpu.SemaphoreType.DMA for _ in range(4)],
            pltpu.SemaphoreType.DMA,
            pltpu.SemaphoreType.DMA,
            pltpu.SemaphoreType.DMA,
        ],
    )
    def scatter_kernel(src2_hbm, dst_hbm, table_hbm, zeros_hbm, out_hbm,
                       srcA, srcB, dst_v, rows, acc, gsem, ssem,
                       semA, semB, semD):
        c = lax.axis_index("c")
        s = lax.axis_index("s")
        r0 = s * _RPT
        pltpu.sync_copy(zeros_hbm.at[pl.ds(r0, _RPT)], acc.at[pl.ds(r0, _RPT)])

        src_bufs = (srcA, srcB)
        idx_sems = (semA, semB)

        def src_hslice(b):
            return src2_hbm.at[c, s, pl.ds(b * BC * CHUNK, BC * CHUNK)]

        def fire_src(b, p):
            pltpu.async_copy(src_hslice(b), src_bufs[p], idx_sems[p])

        def wait_src(b, p):
            pltpu.make_async_copy(src_hslice(b), src_bufs[p],
                                  idx_sems[p]).wait()

        def fire_dst(b):
            pltpu.async_copy(dst_hbm.at[s, b], dst_v, semD)

        def wait_dst(b):
            pltpu.make_async_copy(dst_hbm.at[s, b], dst_v, semD).wait()

        def g_slice(buf, j):
            return buf.at[pl.ds(j * CHUNK, CHUNK)]

        def fire_g(buf, j, k):
            pltpu.async_copy(table_hbm.at[g_slice(buf, j)], rows[k], gsem[k])

        def wait_g(buf, j, k):
            pltpu.make_async_copy(table_hbm.at[g_slice(buf, j)], rows[k],
                                  gsem[k]).wait()

        fire_src(0, 0)
        fire_dst(0)
        plsc.subcore_barrier()
        wait_src(0, 0)
        for k in range(4):
            fire_g(srcA, k, k)

        for b in range(NB):
            p = b % 2
            src_v = src_bufs[p]
            if b + 1 < NB:
                fire_src(b + 1, 1 - p)
            wait_dst(b)

            def wait_s(j, k):
                pltpu.make_async_copy(rows[k], acc.at[dst_v.at[j]],
                                      ssem[k]).wait()

            def body(ii, carry):
                j = 4 * ii
                for k in range(4):
                    wait_g(src_v, j + k, k)
                    kp = (k + 3) % 4
                    if k == 0:
                        @pl.when(ii > 0)
                        def _():
                            fire_g(src_v, j + 3, kp)
                    else:
                        @pl.when(ii < NQ - 1)
                        def _():
                            fire_g(src_v, j + k + 3, kp)

                        if b + 1 < NB:
                            @pl.when(ii == NQ - 1)
                            def _():
                                if k == 1:
                                    wait_src(b + 1, 1 - p)
                                fire_g(src_bufs[1 - p], k - 1, kp)

                    pltpu.sync_copy(rows[k], acc.at[dst_v.at[j + k]],
                                    add=True)
                return carry

            lax.fori_loop(0, NQ, body, 0)
            if b + 1 < NB:
                fire_g(src_bufs[1 - p], 3, 3)
                fire_dst(b + 1)

        plsc.subcore_barrier()
        pltpu.sync_copy(acc.at[pl.ds(r0, _RPT)], out_hbm.at[pl.ds(r0, _RPT), c])

    return scatter_kernel


BRH = BR // 2
BRQ = BR // 4


def _prep_body(deg_ref, x_ref, MA_ref, MB_ref, dv2_ref, xs_ref):
    X = deg_ref[...]
    A = jnp.dot(X, MA_ref[...], preferred_element_type=jnp.float32) + 1.0
    B = jnp.dot(X, MB_ref[...], preferred_element_type=jnp.float32) + 1.0
    dvA = lax.rsqrt(A)
    dvB = lax.rsqrt(B)
    dv2 = jnp.stack([dvA, dvB], axis=1).reshape(BRH, 128)
    dv2_ref[...] = dv2
    xs_ref[...] = dv2 * x_ref[...]


def _tc_prep(degcnt128, xp128, MA, MB):
    return pl.pallas_call(
        _prep_body,
        grid=(G,),
        in_specs=[
            pl.BlockSpec((BRQ, 128), lambda i: (i, 0)),
            pl.BlockSpec((BRH, 128), lambda i: (i, 0)),
            pl.BlockSpec((128, 128), lambda i: (0, 0)),
            pl.BlockSpec((128, 128), lambda i: (0, 0)),
        ],
        out_specs=[
            pl.BlockSpec((BRH, 128), lambda i: (i, 0)),
            pl.BlockSpec((BRH, 128), lambda i: (i, 0)),
        ],
        out_shape=[
            jax.ShapeDtypeStruct((NP // 2, 128), jnp.float32),
            jax.ShapeDtypeStruct((NP // 2, 128), jnp.float32),
        ],
    )(degcnt128, xp128, MA, MB)


def _layer_body(S_ref, xs_ref, dv2_ref, W_ref, scv_ref, shv_ref, out_ref):
    t = S_ref[...] + xs_ref[...]
    z = jnp.dot(t, W_ref[...], preferred_element_type=jnp.float32)
    dv = dv2_ref[...]
    a = jnp.maximum(dv * z * scv_ref[...] + shv_ref[...], 0.0)
    out_ref[...] = dv * a


def _tc_layer(S128, xs_prev, dv2, Wbd, scv, shv):
    return pl.pallas_call(
        _layer_body,
        grid=(G,),
        in_specs=[
            pl.BlockSpec((BRH, 128), lambda i: (i, 0)),
            pl.BlockSpec((BRH, 128), lambda i: (i, 0)),
            pl.BlockSpec((BRH, 128), lambda i: (i, 0)),
            pl.BlockSpec((128, 128), lambda i: (0, 0)),
            pl.BlockSpec((1, 128), lambda i: (0, 0)),
            pl.BlockSpec((1, 128), lambda i: (0, 0)),
        ],
        out_specs=pl.BlockSpec((BRH, 128), lambda i: (i, 0)),
        out_shape=jax.ShapeDtypeStruct((NP // 2, 128), jnp.float32),
    )(S128, xs_prev, dv2, Wbd, scv, shv)


def _final_body(S_ref, xs_ref, dinv_ref, W_ref, scv_ref, shv_ref,
                sW1_ref, sb1_ref, sW2_ref, sb2_ref,
                aW1_ref, ab1_ref, aW2_ref, ab2_ref,
                score_ref, act_ref, acc_ref):
    i = pl.program_id(0)
    t = S_ref[...] + xs_ref[...]
    z = jnp.dot(t, W_ref[...], preferred_element_type=jnp.float32)
    a = jnp.maximum(dinv_ref[...] * z * scv_ref[...] + shv_ref[...], 0.0)
    m = i * BRH + lax.broadcasted_iota(jnp.int32, (BRH, 128), 0)
    node = 2 * m + (lax.broadcasted_iota(jnp.int32, (BRH, 128), 1) >= 64)
    a = jnp.where(node < N, a, 0.0)

    @pl.when(i == 0)
    def _():
        acc_ref[...] = jnp.zeros_like(acc_ref)

    acc_ref[...] += jnp.sum(a, axis=0, keepdims=True)

    @pl.when(i == G - 1)
    def _():
        acc = acc_ref[...]
        emb = (acc[:, :H] + acc[:, H:]) * (1.0 / N)
        h1 = jnp.maximum(
            jnp.dot(emb, sW1_ref[...], preferred_element_type=jnp.float32)
            + sb1_ref[...], 0.0)
        sc = jnp.dot(h1, sW2_ref[...], preferred_element_type=jnp.float32) \
            + sb2_ref[...]
        score_ref[...] = 1.0 / (1.0 + jnp.exp(-sc))
        h2 = jnp.maximum(
            jnp.dot(emb, aW1_ref[...], preferred_element_type=jnp.float32)
            + ab1_ref[...], 0.0)
        act_ref[...] = jnp.dot(h2, aW2_ref[...],
                               preferred_element_type=jnp.float32) + ab2_ref[...]


def _tc_final(S128, xs_prev, dv2, Wbd, scv, shv, sW1, sb1, sW2, sb2,
              aW1, ab1, aW2, ab2):
    full = lambda r, c: pl.BlockSpec((r, c), lambda i: (0, 0))
    return pl.pallas_call(
        _final_body,
        grid=(G,),
        in_specs=[
            pl.BlockSpec((BRH, 128), lambda i: (i, 0)),
            pl.BlockSpec((BRH, 128), lambda i: (i, 0)),
            pl.BlockSpec((BRH, 128), lambda i: (i, 0)),
            full(128, 128), full(1, 128), full(1, 128),
            full(H, 32), full(1, 32), full(32, 1), full(1, 1),
            full(H, 32), full(1, 32), full(32, 4), full(1, 4),
        ],
        out_specs=[full(1, 1), full(1, 4)],
        out_shape=[
            jax.ShapeDtypeStruct((1, 1), jnp.float32),
            jax.ShapeDtypeStruct((1, 4), jnp.float32),
        ],
        scratch_shapes=[pltpu.VMEM((1, 128), jnp.float32)],
    )(S128, xs_prev, dv2, Wbd, scv, shv, sW1, sb1, sW2, sb2,
      aW1, ab1, aW2, ab2)


_deg_kernel = _make_deg_kernel()
_scatter32 = _make_scatter_kernel(32)


def _blockdiag2(W64):
    return jnp.zeros((128, 128), jnp.float32).at[:H, :H].set(
        W64).at[H:, H:].set(W64)


def kernel(x, edge_index, batch, W1, b1, W2, b2, W3, b3, g1, be1, g2, be2,
           g3, be3, sW1, sb1, sW2, sb2, aW1, ab1, aW2, ab2):
    f32 = jnp.float32
    src = edge_index[0].astype(jnp.int32)
    dst = edge_index[1].astype(jnp.int32)

    npad = EP - E
    pad_i = jnp.arange(npad, dtype=jnp.int32)
    src_p = jnp.concatenate([src, pad_i % N])
    dst_p = jnp.concatenate([dst, N + (pad_i % (NP - N - 1))])
    src2_flat = src_p * 2
    src2 = jnp.stack([src2_flat, src2_flat + 1]).reshape(
        2, _SC_TILES, (NCH // _SC_TILES) * CHUNK)
    dstc = dst_p.reshape(_SC_TILES, (NCH // _SC_TILES) // 28, 28, CHUNK)
    dstc_deg = dst_p.reshape(32, NCH // 32, CHUNK)

    ones16 = jnp.ones((CHUNK, 16), f32)
    zeros16 = jnp.zeros((NP, 16), f32)
    zeros32 = jnp.zeros((NP, 32), f32)

    q = 1.0 / jnp.sqrt(1.0 + BN_EPS)

    def fold(g, b, be):
        scv = jnp.tile((g * q).reshape(1, H), (1, 2))
        shv = jnp.tile((b * g * q + be).reshape(1, H), (1, 2))
        return scv, shv

    scv1, shv1 = fold(g1, b1, be1)
    scv2, shv2 = fold(g2, b2, be2)
    scv3, shv3 = fold(g3, b3, be3)
    W1bd = _blockdiag2(jnp.zeros((H, H), f32).at[:20].set(W1))
    W2bd = _blockdiag2(W2)
    W3bd = _blockdiag2(W3)

    cols = jnp.arange(128)
    rows = jnp.arange(128)[:, None]
    bA = (cols >= 64).astype(jnp.int32)
    MA = ((rows == 32 * bA) | (rows == 32 * bA + 16)).astype(f32)
    bB = bA + 2
    MB = ((rows == 32 * bB) | (rows == 32 * bB + 16)).astype(f32)

    xp128 = jnp.zeros((NP, H), f32).at[:N, :20].set(x).reshape(NP // 2, 128)

    degcnt = _deg_kernel(dstc_deg, ones16, zeros16)
    dv2, xs1 = _tc_prep(degcnt.reshape(NP // 4, 128), xp128, MA, MB)

    S1 = _scatter32(src2, dstc, xs1.reshape(2 * NP, 32), zeros32)
    xs2 = _tc_layer(S1.reshape(NP // 2, 128), xs1, dv2, W1bd, scv1, shv1)

    S2 = _scatter32(src2, dstc, xs2.reshape(2 * NP, 32), zeros32)
    xs3 = _tc_layer(S2.reshape(NP // 2, 128), xs2, dv2, W2bd, scv2, shv2)

    S3 = _scatter32(src2, dstc, xs3.reshape(2 * NP, 32), zeros32)
    score, actions = _tc_final(
        S3.reshape(NP // 2, 128), xs3, dv2, W3bd, scv3, shv3,
        sW1, sb1.reshape(1, 32), sW2, sb2.reshape(1, 1),
        aW1, ab1.reshape(1, 32), aW2, ab2.reshape(1, 4))
    return (score, actions)
